# baseline (speedup 1.0000x reference)
import functools
import math

import jax
import jax.numpy as jnp
from jax import lax
from jax.experimental import pallas as pl
from jax.experimental.pallas import tpu as pltpu
from jax.experimental.pallas import tpu_sc as plsc

_W_ORI = 1285
_H_ORI = 438
_FARO_V = 123.5
_FARO_H = 360.0
_CROP = 384

_N = 128 * 128
_SPLIT = 15360
_TA = 512
_TB = 2048
_NI = _SPLIT // _TA
_NJ = _N // _TB
_K = 8

_NW = 32
_L = 16


def _directions(h, w, sh, sw):
    fv = _FARO_V * _CROP / _H_ORI
    fh = _FARO_H * _CROP / _W_ORI
    cw_rad = sw / _W_ORI * _FARO_H
    ch_rad = sh / _H_ORI * _FARO_V
    p, q = jnp.meshgrid(jnp.arange(h), jnp.arange(w), indexing="ij")
    points_hw = jnp.stack([p, q], axis=-1).reshape(-1, 2).astype(jnp.float32)
    yaw = (-fh * points_hw[:, 1] / w + cw_rad) * (math.pi / 180.0)
    pitch = (-fv * points_hw[:, 0] / h + ch_rad) * (math.pi / 180.0)
    ux = jnp.sin(yaw) * jnp.sin(pitch)
    uy = jnp.cos(yaw) * jnp.sin(pitch)
    uz = jnp.cos(pitch)
    return ux, uy, uz


def _chamfer_body(a_ref, p2_ref, b_ref, q2_ref, d2_ref, s1_ref):
    i = pl.program_id(0)

    @pl.when(i == 0)
    def _init():
        d2_ref[...] = jnp.full((1, _N), jnp.inf, jnp.float32)
        s1_ref[0] = 0.0

    a_blk = a_ref[...]
    p2_blk = p2_ref[...]

    rowmin = jnp.full((_TA, 1), jnp.inf, jnp.float32)
    for j in range(_NJ):
        cols = pl.ds(j * _TB, _TB)
        ab2 = jax.lax.dot_general(
            a_blk, b_ref[:, cols],
            (((1,), (0,)), ((), ())),
            preferred_element_type=jnp.float32)
        rowmin = jnp.minimum(
            rowmin, jnp.min(ab2 + q2_ref[:, cols], axis=1, keepdims=True))
        d2_ref[:, cols] = jnp.minimum(
            d2_ref[:, cols], jnp.min(ab2 + p2_blk, axis=0, keepdims=True))
    s1_ref[0] += jnp.sum(rowmin + p2_blk)


def _tc_chamfer_call(a_mat, p2_col, b_mat, q2_row):
    return pl.pallas_call(
        _chamfer_body,
        grid=(_NI,),
        in_specs=[
            pl.BlockSpec((_TA, _K), lambda i: (i, 0)),
            pl.BlockSpec((_TA, 1), lambda i: (i, 0)),
            pl.BlockSpec((_K, _N), lambda i: (0, 0)),
            pl.BlockSpec((1, _N), lambda i: (0, 0)),
        ],
        out_specs=[
            pl.BlockSpec((1, _N), lambda i: (0, 0)),
            pl.BlockSpec(memory_space=pltpu.SMEM),
        ],
        out_shape=[
            jax.ShapeDtypeStruct((1, _N), jnp.float32),
            jax.ShapeDtypeStruct((1,), jnp.float32),
        ],
    )(a_mat, p2_col, b_mat, q2_row)


def _sc_chamfer_call(ax, ay, az, p2, bx, by, bz, q2):
    rw = ax.shape[0] // _NW
    mesh = plsc.VectorSubcoreMesh(core_axis_name="c", subcore_axis_name="s")

    @functools.partial(
        pl.kernel, mesh=mesh,
        out_type=[jax.ShapeDtypeStruct((_NW, _L), jnp.float32),
                  jax.ShapeDtypeStruct((_NW, _N), jnp.float32)],
        scratch_types=[
            pltpu.VMEM((rw,), jnp.float32),
            pltpu.VMEM((rw,), jnp.float32),
            pltpu.VMEM((rw,), jnp.float32),
            pltpu.VMEM((rw,), jnp.float32),
            pltpu.VMEM((_N,), jnp.float32),
            pltpu.VMEM((_N,), jnp.float32),
            pltpu.VMEM((_N,), jnp.float32),
            pltpu.VMEM((_N,), jnp.float32),
            pltpu.VMEM((_L,), jnp.float32),
            pltpu.VMEM((_N,), jnp.float32),
        ],
    )
    def sc_chamfer(ax_h, ay_h, az_h, p2_h, bx_h, by_h, bz_h, q2_h,
                   sum_out, d2_out,
                   axv, ayv, azv, p2v, bxv, byv, bzv, q2v, sumv, d2v):
        wid = lax.axis_index("s") * 2 + lax.axis_index("c")
        base = wid * rw
        pltpu.sync_copy(ax_h.at[pl.ds(base, rw)], axv)
        pltpu.sync_copy(ay_h.at[pl.ds(base, rw)], ayv)
        pltpu.sync_copy(az_h.at[pl.ds(base, rw)], azv)
        pltpu.sync_copy(p2_h.at[pl.ds(base, rw)], p2v)
        pltpu.sync_copy(bx_h, bxv)
        pltpu.sync_copy(by_h, byv)
        pltpu.sync_copy(bz_h, bzv)
        pltpu.sync_copy(q2_h, q2v)

        def row_group(ig, sacc):
            rows = pl.ds(ig * _L, _L)
            pxg = axv[rows]
            pyg = ayv[rows]
            pzg = azv[rows]
            p2g = p2v[rows]

            def jstep(j, rm):
                grp = pl.ds(j * _L, _L)
                bxg = bxv[grp]
                byg = byv[grp]
                bzg = bzv[grp]
                q2g = q2v[grp]
                for u in range(_L):
                    ab2 = (pxg * bxg[u] + pyg * byg[u]) + pzg * bzg[u]
                    rm = jnp.minimum(rm, ab2 + q2g[u])
                return rm

            rm = lax.fori_loop(0, _N // _L, jstep,
                               jnp.full((_L,), jnp.inf, jnp.float32))
            return sacc + (p2g + rm)

        sumv[...] = lax.fori_loop(0, rw // _L, row_group,
                                  jnp.zeros((_L,), jnp.float32))
        pltpu.sync_copy(sumv, sum_out.at[wid])

        def col_group(jg, carry):
            cols = pl.ds(jg * _L, _L)
            qxg = bxv[cols]
            qyg = byv[cols]
            qzg = bzv[cols]

            def istep(i, dm):
                grp = pl.ds(i * _L, _L)
                axg = axv[grp]
                ayg = ayv[grp]
                azg = azv[grp]
                p2g = p2v[grp]
                for u in range(_L):
                    ab2 = (qxg * axg[u] + qyg * ayg[u]) + qzg * azg[u]
                    dm = jnp.minimum(dm, ab2 + p2g[u])
                return dm

            dm = lax.fori_loop(0, rw // _L, istep,
                               jnp.full((_L,), jnp.inf, jnp.float32))
            d2v[cols] = dm
            return carry

        lax.fori_loop(0, _N // _L, col_group, 0)
        pltpu.sync_copy(d2v, d2_out.at[wid])

    return sc_chamfer(ax, ay, az, p2, bx, by, bz, q2)


def kernel(fake, tar, sh, sw):
    b, _, h, w = fake.shape
    ux, uy, uz = _directions(h, w, sh, sw)
    dp = tar[b - 1, 0].reshape(-1)
    dq = fake[b - 1, 0].reshape(-1)

    px, py, pz = dp * ux, dp * uy, dp * uz
    qx, qy, qz = dq * ux, dq * uy, dq * uz
    p2 = px * px + py * py + pz * pz
    q2 = qx * qx + qy * qy + qz * qz
    m2 = jnp.float32(-2.0)
    ax, ay, az = m2 * px, m2 * py, m2 * pz
    zero = jnp.zeros((_N,), jnp.float32)
    a_mat = jnp.stack([ax, ay, az, zero, zero, zero, zero, zero],
                      axis=1).astype(jnp.bfloat16)
    b_mat = jnp.stack([qx, qy, qz, zero, zero, zero, zero, zero],
                      axis=0).astype(jnp.bfloat16)

    def _bf(x):
        return lax.reduce_precision(x, 8, 7)

    sc_sum, sc_d2 = _sc_chamfer_call(
        _bf(ax[_SPLIT:]), _bf(ay[_SPLIT:]), _bf(az[_SPLIT:]), p2[_SPLIT:],
        _bf(qx), _bf(qy), _bf(qz), q2)
    tc_d2, tc_s1 = _tc_chamfer_call(
        a_mat[:_SPLIT], p2[:_SPLIT].reshape(_SPLIT, 1),
        b_mat, q2.reshape(1, _N))

    d2 = jnp.minimum(tc_d2[0], jnp.min(sc_d2, axis=0))
    loss = (tc_s1[0] + jnp.sum(sc_sum) + jnp.sum(d2 + q2)) / float(_N)
    return loss


# speedup vs baseline: 1.1902x; 1.1902x over previous
import functools
import math

import jax
import jax.numpy as jnp
from jax import lax
from jax.experimental import pallas as pl
from jax.experimental.pallas import tpu as pltpu
from jax.experimental.pallas import tpu_sc as plsc

_W_ORI = 1285
_H_ORI = 438
_FARO_V = 123.5
_FARO_H = 360.0
_CROP = 384

_N = 128 * 128
_SPLIT = 15872
_TA = 512
_TB = 2048
_NI = _SPLIT // _TA
_NJ = _N // _TB
_K = 8

_NW = 32
_L = 16


def _directions(h, w, sh, sw):
    fv = _FARO_V * _CROP / _H_ORI
    fh = _FARO_H * _CROP / _W_ORI
    cw_rad = sw / _W_ORI * _FARO_H
    ch_rad = sh / _H_ORI * _FARO_V
    p, q = jnp.meshgrid(jnp.arange(h), jnp.arange(w), indexing="ij")
    points_hw = jnp.stack([p, q], axis=-1).reshape(-1, 2).astype(jnp.float32)
    yaw = (-fh * points_hw[:, 1] / w + cw_rad) * (math.pi / 180.0)
    pitch = (-fv * points_hw[:, 0] / h + ch_rad) * (math.pi / 180.0)
    ux = jnp.sin(yaw) * jnp.sin(pitch)
    uy = jnp.cos(yaw) * jnp.sin(pitch)
    uz = jnp.cos(pitch)
    return ux, uy, uz


def _chamfer_body(a_ref, p2_ref, b_ref, q2_ref, d2_ref, s1_ref):
    i = pl.program_id(0)

    @pl.when(i == 0)
    def _init():
        d2_ref[...] = jnp.full((1, _N), jnp.inf, jnp.float32)
        s1_ref[0] = 0.0

    a_blk = a_ref[...]
    p2_blk = p2_ref[...]

    rowmin = jnp.full((_TA, 1), jnp.inf, jnp.float32)
    for j in range(_NJ):
        cols = pl.ds(j * _TB, _TB)
        ab2 = jax.lax.dot_general(
            a_blk, b_ref[:, cols],
            (((1,), (0,)), ((), ())),
            preferred_element_type=jnp.float32)
        rowmin = jnp.minimum(
            rowmin, jnp.min(ab2 + q2_ref[:, cols], axis=1, keepdims=True))
        d2_ref[:, cols] = jnp.minimum(
            d2_ref[:, cols], jnp.min(ab2 + p2_blk, axis=0, keepdims=True))
    s1_ref[0] += jnp.sum(rowmin + p2_blk)


def _tc_chamfer_call(a_mat, p2_col, b_mat, q2_row):
    return pl.pallas_call(
        _chamfer_body,
        grid=(_NI,),
        in_specs=[
            pl.BlockSpec((_TA, _K), lambda i: (i, 0)),
            pl.BlockSpec((_TA, 1), lambda i: (i, 0)),
            pl.BlockSpec((_K, _N), lambda i: (0, 0)),
            pl.BlockSpec((1, _N), lambda i: (0, 0)),
        ],
        out_specs=[
            pl.BlockSpec((1, _N), lambda i: (0, 0)),
            pl.BlockSpec(memory_space=pltpu.SMEM),
        ],
        out_shape=[
            jax.ShapeDtypeStruct((1, _N), jnp.float32),
            jax.ShapeDtypeStruct((1,), jnp.float32),
        ],
    )(a_mat, p2_col, b_mat, q2_row)


def _sc_chamfer_call(ax, ay, az, p2, bx, by, bz, q2):
    rw = ax.shape[0] // _NW
    mesh = plsc.VectorSubcoreMesh(core_axis_name="c", subcore_axis_name="s")

    @functools.partial(
        pl.kernel, mesh=mesh,
        out_type=[jax.ShapeDtypeStruct((_NW, _L), jnp.float32),
                  jax.ShapeDtypeStruct((_NW, _N), jnp.float32)],
        scratch_types=[
            pltpu.VMEM((rw,), jnp.float32),
            pltpu.VMEM((rw,), jnp.float32),
            pltpu.VMEM((rw,), jnp.float32),
            pltpu.VMEM((rw,), jnp.float32),
            pltpu.VMEM((_N,), jnp.float32),
            pltpu.VMEM((_N,), jnp.float32),
            pltpu.VMEM((_N,), jnp.float32),
            pltpu.VMEM((_N,), jnp.float32),
            pltpu.VMEM((_L,), jnp.float32),
            pltpu.VMEM((_N,), jnp.float32),
        ],
    )
    def sc_chamfer(ax_h, ay_h, az_h, p2_h, bx_h, by_h, bz_h, q2_h,
                   sum_out, d2_out,
                   axv, ayv, azv, p2v, bxv, byv, bzv, q2v, sumv, d2v):
        wid = lax.axis_index("s") * 2 + lax.axis_index("c")
        base = wid * rw
        pltpu.sync_copy(ax_h.at[pl.ds(base, rw)], axv)
        pltpu.sync_copy(ay_h.at[pl.ds(base, rw)], ayv)
        pltpu.sync_copy(az_h.at[pl.ds(base, rw)], azv)
        pltpu.sync_copy(p2_h.at[pl.ds(base, rw)], p2v)
        pltpu.sync_copy(bx_h, bxv)
        pltpu.sync_copy(by_h, byv)
        pltpu.sync_copy(bz_h, bzv)
        pltpu.sync_copy(q2_h, q2v)

        def row_group(ig, sacc):
            rows = pl.ds(ig * _L, _L)
            pxg = axv[rows]
            pyg = ayv[rows]
            pzg = azv[rows]
            p2g = p2v[rows]

            def jstep(j, rm):
                grp = pl.ds(j * _L, _L)
                bxg = bxv[grp]
                byg = byv[grp]
                bzg = bzv[grp]
                q2g = q2v[grp]
                for u in range(_L):
                    ab2 = (pxg * bxg[u] + pyg * byg[u]) + pzg * bzg[u]
                    rm = jnp.minimum(rm, ab2 + q2g[u])
                return rm

            rm = lax.fori_loop(0, _N // _L, jstep,
                               jnp.full((_L,), jnp.inf, jnp.float32))
            return sacc + (p2g + rm)

        sumv[...] = lax.fori_loop(0, rw // _L, row_group,
                                  jnp.zeros((_L,), jnp.float32))
        pltpu.sync_copy(sumv, sum_out.at[wid])

        def col_group(jg, carry):
            cols = pl.ds(jg * _L, _L)
            qxg = bxv[cols]
            qyg = byv[cols]
            qzg = bzv[cols]

            def istep(i, dm):
                grp = pl.ds(i * _L, _L)
                axg = axv[grp]
                ayg = ayv[grp]
                azg = azv[grp]
                p2g = p2v[grp]
                for u in range(_L):
                    ab2 = (qxg * axg[u] + qyg * ayg[u]) + qzg * azg[u]
                    dm = jnp.minimum(dm, ab2 + p2g[u])
                return dm

            dm = lax.fori_loop(0, rw // _L, istep,
                               jnp.full((_L,), jnp.inf, jnp.float32))
            d2v[cols] = dm
            return carry

        lax.fori_loop(0, _N // _L, col_group, 0)
        pltpu.sync_copy(d2v, d2_out.at[wid])

    return sc_chamfer(ax, ay, az, p2, bx, by, bz, q2)


def kernel(fake, tar, sh, sw):
    b, _, h, w = fake.shape
    ux, uy, uz = _directions(h, w, sh, sw)
    dp = tar[b - 1, 0].reshape(-1)
    dq = fake[b - 1, 0].reshape(-1)

    px, py, pz = dp * ux, dp * uy, dp * uz
    qx, qy, qz = dq * ux, dq * uy, dq * uz
    p2 = px * px + py * py + pz * pz
    q2 = qx * qx + qy * qy + qz * qz
    m2 = jnp.float32(-2.0)
    ax, ay, az = m2 * px, m2 * py, m2 * pz
    zero = jnp.zeros((_N,), jnp.float32)
    a_mat = jnp.stack([ax, ay, az, zero, zero, zero, zero, zero],
                      axis=1).astype(jnp.bfloat16)
    b_mat = jnp.stack([qx, qy, qz, zero, zero, zero, zero, zero],
                      axis=0).astype(jnp.bfloat16)

    def _bf(x):
        return lax.reduce_precision(x, 8, 7)

    sc_sum, sc_d2 = _sc_chamfer_call(
        _bf(ax[_SPLIT:]), _bf(ay[_SPLIT:]), _bf(az[_SPLIT:]), p2[_SPLIT:],
        _bf(qx), _bf(qy), _bf(qz), q2)
    tc_d2, tc_s1 = _tc_chamfer_call(
        a_mat[:_SPLIT], p2[:_SPLIT].reshape(_SPLIT, 1),
        b_mat, q2.reshape(1, _N))

    d2 = jnp.minimum(tc_d2[0], jnp.min(sc_d2, axis=0))
    loss = (tc_s1[0] + jnp.sum(sc_sum) + jnp.sum(d2 + q2)) / float(_N)
    return loss
